# baseline (speedup 1.0000x reference)
import functools

import jax
import jax.numpy as jnp
from jax import lax
from jax.experimental import pallas as pl
from jax.experimental.pallas import tpu as pltpu
from jax.experimental.pallas import tpu_sc as plsc

_LANES = 16
_GATHER_CHUNK = 128


@jax.jit
def _sc_embed_call(ids, ptab, ttab, p2t):
    B = ids.shape[0]
    V, D = ptab.shape
    ttab_shape = ttab.shape

    info = plsc.get_sparse_core_info()
    NC, NS = info.num_cores, info.num_subcores
    NW = NC * NS
    bpw = B // NW
    n_ch = bpw // _GATHER_CHUNK

    mesh = plsc.VectorSubcoreMesh(core_axis_name="c", subcore_axis_name="s")

    @functools.partial(
        pl.kernel,
        mesh=mesh,
        compiler_params=pltpu.CompilerParams(use_tc_tiling_on_sc=False),
        out_type=jax.ShapeDtypeStruct((B, D), jnp.float32),
        scratch_types=[
            pltpu.VMEM((bpw,), jnp.int32),
            pltpu.VMEM((bpw,), jnp.int32),
            pltpu.VMEM((bpw, D), jnp.float32),
            pltpu.VMEM(ttab_shape, jnp.float32),
            pltpu.SemaphoreType.DMA,
            pltpu.SemaphoreType.DMA,
        ],
    )
    def sc_embed(pid_hbm, ptab_hbm, ttab_hbm, p2t_hbm, out_hbm,
                 idx_v, tids_v, rows_v, ttab_v, sem_rows, sem_tids):
        wid = lax.axis_index("s") * NC + lax.axis_index("c")
        base = wid * bpw
        pltpu.sync_copy(pid_hbm.at[pl.ds(base, bpw)], idx_v)

        row_copies = []
        tid_copies = []
        for c in range(n_ch):
            sl = pl.ds(c * _GATHER_CHUNK, _GATHER_CHUNK)
            row_copies.append(
                pltpu.async_copy(ptab_hbm.at[idx_v.at[sl]], rows_v.at[sl],
                                 sem_rows))
            tid_copies.append(
                pltpu.async_copy(p2t_hbm.at[idx_v.at[sl]], tids_v.at[sl],
                                 sem_tids))
        pltpu.sync_copy(ttab_hbm, ttab_v)
        for cp in tid_copies:
            cp.wait()
        for cp in row_copies:
            cp.wait()

        @pl.loop(0, bpw, step=_LANES)
        def _(i):
            tid16 = tids_v[pl.ds(i, _LANES)]
            for k in range(_LANES):
                t = tid16[k]
                for j in range(0, D, _LANES):
                    sl = pl.ds(j, _LANES)
                    rows_v[i + k, sl] += ttab_v[t, sl]

        pltpu.sync_copy(rows_v, out_hbm.at[pl.ds(base, bpw)])

    return sc_embed(ids, ptab, ttab, p2t)


def kernel(primitive_ids, primitive_embeddings, type_embeddings,
           primitive_to_type):
    ids = primitive_ids.astype(jnp.int32)
    p2t = primitive_to_type.astype(jnp.int32)
    return _sc_embed_call(ids, primitive_embeddings, type_embeddings, p2t)


# speedup vs baseline: 1.3448x; 1.3448x over previous
import functools

import jax
import jax.numpy as jnp
from jax import lax
from jax.experimental import pallas as pl
from jax.experimental.pallas import tpu as pltpu
from jax.experimental.pallas import tpu_sc as plsc

_LANES = 16
_GATHER_CHUNK = 128


@jax.jit
def _sc_embed_call(ids, ptab, ttab, p2t):
    B = ids.shape[0]
    V, D = ptab.shape
    ttab_shape = ttab.shape

    info = plsc.get_sparse_core_info()
    NC, NS = info.num_cores, info.num_subcores
    NW = NC * NS
    bpw = B // NW
    n_ch = bpw // _GATHER_CHUNK

    mesh = plsc.VectorSubcoreMesh(core_axis_name="c", subcore_axis_name="s")

    @functools.partial(
        pl.kernel,
        mesh=mesh,
        compiler_params=pltpu.CompilerParams(use_tc_tiling_on_sc=True),
        out_type=jax.ShapeDtypeStruct((B, D), jnp.float32),
        scratch_types=[
            pltpu.VMEM((bpw,), jnp.int32),
            pltpu.VMEM((bpw,), jnp.int32),
            pltpu.VMEM((bpw, D), jnp.float32),
            pltpu.VMEM(ttab_shape, jnp.float32),
            pltpu.SemaphoreType.DMA,
            pltpu.SemaphoreType.DMA,
        ],
    )
    def sc_embed(pid_hbm, ptab_hbm, ttab_hbm, p2t_hbm, out_hbm,
                 idx_v, tids_v, rows_v, ttab_v, sem_rows, sem_tids):
        wid = lax.axis_index("s") * NC + lax.axis_index("c")
        base = wid * bpw
        pltpu.sync_copy(pid_hbm.at[pl.ds(base, bpw)], idx_v)

        tid_copies = []
        for c in range(n_ch):
            sl = pl.ds(c * _GATHER_CHUNK, _GATHER_CHUNK)
            tid_copies.append(
                pltpu.async_copy(p2t_hbm.at[idx_v.at[sl]], tids_v.at[sl],
                                 sem_tids))
        pltpu.sync_copy(ttab_hbm, ttab_v)

        @pl.loop(0, bpw, step=_LANES)
        def _(i):
            id16 = idx_v[pl.ds(i, _LANES)]
            for k in range(_LANES):
                pltpu.async_copy(ptab_hbm.at[id16[k]], rows_v.at[i + k],
                                 sem_rows)

        for cp in tid_copies:
            cp.wait()
        pltpu.make_async_copy(out_hbm.at[pl.ds(base, bpw)], rows_v,
                              sem_rows).wait()

        @pl.loop(0, bpw, step=_LANES)
        def _(i):
            tid16 = tids_v[pl.ds(i, _LANES)]
            for k in range(_LANES):
                t = tid16[k]
                for j in range(0, D, _LANES):
                    sl = pl.ds(j, _LANES)
                    rows_v[i + k, sl] += ttab_v[t, sl]

        pltpu.sync_copy(rows_v, out_hbm.at[pl.ds(base, bpw)])

    return sc_embed(ids, ptab, ttab, p2t)


def kernel(primitive_ids, primitive_embeddings, type_embeddings,
           primitive_to_type):
    ids = primitive_ids.astype(jnp.int32)
    p2t = primitive_to_type.astype(jnp.int32)
    return _sc_embed_call(ids, primitive_embeddings, type_embeddings, p2t)


# speedup vs baseline: 1.3775x; 1.0243x over previous
import functools

import jax
import jax.numpy as jnp
from jax import lax
from jax.experimental import pallas as pl
from jax.experimental.pallas import tpu as pltpu
from jax.experimental.pallas import tpu_sc as plsc

_LANES = 16
_CHUNK = 128


@jax.jit
def _sc_embed_call(ids, ptab, ttab, p2t):
    B = ids.shape[0]
    V, D = ptab.shape
    ttab_shape = ttab.shape

    info = plsc.get_sparse_core_info()
    NC, NS = info.num_cores, info.num_subcores
    NW = NC * NS
    bpw = B // NW
    n_ch = bpw // _CHUNK

    mesh = plsc.VectorSubcoreMesh(core_axis_name="c", subcore_axis_name="s")

    @functools.partial(
        pl.kernel,
        mesh=mesh,
        compiler_params=pltpu.CompilerParams(use_tc_tiling_on_sc=True),
        out_type=jax.ShapeDtypeStruct((B, D), jnp.float32),
        scratch_types=[
            pltpu.VMEM((bpw,), jnp.int32),
            pltpu.VMEM((bpw,), jnp.int32),
            pltpu.VMEM((bpw, D), jnp.float32),
            pltpu.VMEM(ttab_shape, jnp.float32),
            pltpu.SemaphoreType.DMA,
            pltpu.SemaphoreType.DMA,
            [pltpu.SemaphoreType.DMA] * 4,
        ],
    )
    def sc_embed(pid_hbm, ptab_hbm, ttab_hbm, p2t_hbm, out_hbm,
                 idx_v, tids_v, rows_v, ttab_v, sem_tids, sem_out, sem_rows):
        wid = lax.axis_index("s") * NC + lax.axis_index("c")
        base = wid * bpw
        pltpu.sync_copy(pid_hbm.at[pl.ds(base, bpw)], idx_v)

        tid_copies = []
        for c in range(n_ch):
            sl = pl.ds(c * _CHUNK, _CHUNK)
            tid_copies.append(
                pltpu.async_copy(p2t_hbm.at[idx_v.at[sl]], tids_v.at[sl],
                                 sem_tids))

        for c in range(n_ch):
            @pl.loop(c * _CHUNK, (c + 1) * _CHUNK, step=_LANES)
            def _(i, _c=c):
                id16 = idx_v[pl.ds(i, _LANES)]
                for k in range(_LANES):
                    pltpu.async_copy(ptab_hbm.at[id16[k]], rows_v.at[i + k],
                                     sem_rows[_c])

        pltpu.sync_copy(ttab_hbm, ttab_v)
        for cp in tid_copies:
            cp.wait()

        out_copies = []
        for c in range(n_ch):
            sl = pl.ds(c * _CHUNK, _CHUNK)
            pltpu.make_async_copy(out_hbm.at[pl.ds(base, _CHUNK)],
                                  rows_v.at[sl], sem_rows[c]).wait()

            @pl.loop(c * _CHUNK, (c + 1) * _CHUNK, step=_LANES)
            def _(i):
                tid16 = tids_v[pl.ds(i, _LANES)]
                for k in range(_LANES):
                    t = tid16[k]
                    for j in range(0, D, _LANES):
                        sj = pl.ds(j, _LANES)
                        rows_v[i + k, sj] += ttab_v[t, sj]

            out_copies.append(
                pltpu.async_copy(rows_v.at[sl],
                                 out_hbm.at[pl.ds(base + c * _CHUNK, _CHUNK)],
                                 sem_out))
        for cp in out_copies:
            cp.wait()

    return sc_embed(ids, ptab, ttab, p2t)


def kernel(primitive_ids, primitive_embeddings, type_embeddings,
           primitive_to_type):
    ids = primitive_ids.astype(jnp.int32)
    p2t = primitive_to_type.astype(jnp.int32)
    return _sc_embed_call(ids, primitive_embeddings, type_embeddings, p2t)
